# baseline (speedup 1.0000x reference)
import functools

import jax
import jax.numpy as jnp
from jax import lax
from jax.experimental import pallas as pl
from jax.experimental.pallas import tpu as pltpu
from jax.experimental.pallas import tpu_sc as plsc

_BATCH = 16384
_DIM = 32
_NW = 32
_BPW = _BATCH // _NW
_GROUPS = _BPW // 16


def _body(samples_hbm, user_emb, game_emb, user_bias, game_bias, out_hbm,
          samp_v, idxu_v, idxg_v, idxub_v, idxgb_v,
          urows_v, grows_v, ubrows_v, gbrows_v, out_v, sem):
    wid = lax.axis_index("s") * 2 + lax.axis_index("c")
    base = wid * _BPW

    pltpu.sync_copy(samples_hbm.at[pl.ds(base, _BPW)], samp_v)

    def deint(i, _):
        ids = lax.iota(jnp.int32, 16) + i * 16
        zero = jnp.zeros((16,), jnp.int32)
        one = jnp.ones((16,), jnp.int32)
        u = plsc.load_gather(samp_v, [ids, zero])
        g = plsc.load_gather(samp_v, [ids, one])
        idxu_v[pl.ds(i * 16, 16)] = u
        idxg_v[pl.ds(i * 16, 16)] = g
        idxub_v[pl.ds(i * 16, 16)] = u >> 4
        idxgb_v[pl.ds(i * 16, 16)] = g >> 4
        return 0
    lax.fori_loop(0, _GROUPS, deint, 0)

    c1 = pltpu.async_copy(user_emb.at[idxu_v], urows_v, sem)
    c2 = pltpu.async_copy(game_emb.at[idxg_v], grows_v, sem)
    c3 = pltpu.async_copy(user_bias.at[idxub_v], ubrows_v, sem)
    c4 = pltpu.async_copy(game_bias.at[idxgb_v], gbrows_v, sem)
    c1.wait()
    c2.wait()
    c3.wait()
    c4.wait()

    def dot_group(i, _):
        ids = lax.iota(jnp.int32, 16) + i * 16
        lane_u = idxu_v[pl.ds(i * 16, 16)] & 15
        lane_g = idxg_v[pl.ds(i * 16, 16)] & 15
        acc = plsc.load_gather(ubrows_v, [ids, lane_u]) + \
            plsc.load_gather(gbrows_v, [ids, lane_g])
        for d in range(_DIM):
            dd = jnp.full((16,), d, jnp.int32)
            acc = acc + (plsc.load_gather(urows_v, [ids, dd]) *
                         plsc.load_gather(grows_v, [ids, dd]))
        out_v[pl.ds(i * 16, 16)] = acc
        return 0
    lax.fori_loop(0, _GROUPS, dot_group, 0)

    pltpu.sync_copy(out_v, out_hbm.at[pl.ds(base, _BPW)])


@jax.jit
def _run(samples, user_emb, game_emb, user_bias16, game_bias16):
    mesh = plsc.VectorSubcoreMesh(core_axis_name="c", subcore_axis_name="s")
    f = functools.partial(
        pl.kernel,
        mesh=mesh,
        out_type=jax.ShapeDtypeStruct((_BATCH,), jnp.float32),
        compiler_params=pltpu.CompilerParams(
            needs_layout_passes=False, use_tc_tiling_on_sc=False),
        scratch_types=[
            pltpu.VMEM((_BPW, 2), jnp.int32),
            pltpu.VMEM((_BPW,), jnp.int32),
            pltpu.VMEM((_BPW,), jnp.int32),
            pltpu.VMEM((_BPW,), jnp.int32),
            pltpu.VMEM((_BPW,), jnp.int32),
            pltpu.VMEM((_BPW, _DIM), jnp.float32),
            pltpu.VMEM((_BPW, _DIM), jnp.float32),
            pltpu.VMEM((_BPW, 16), jnp.float32),
            pltpu.VMEM((_BPW, 16), jnp.float32),
            pltpu.VMEM((_BPW,), jnp.float32),
            pltpu.SemaphoreType.DMA,
        ],
    )(_body)
    return f(samples, user_emb, game_emb, user_bias16, game_bias16)


def kernel(samples, user_emb, game_emb, user_bias, game_bias):
    samples_i32 = samples.astype(jnp.int32)
    ub16 = user_bias.reshape(user_bias.shape[0] // 16, 16)
    gb16 = game_bias.reshape(game_bias.shape[0] // 16, 16)
    return _run(samples_i32, user_emb, game_emb, ub16, gb16)


# speedup vs baseline: 1.0170x; 1.0170x over previous
import functools

import jax
import jax.numpy as jnp
from jax import lax
from jax.experimental import pallas as pl
from jax.experimental.pallas import tpu as pltpu
from jax.experimental.pallas import tpu_sc as plsc

_BATCH = 16384
_DIM = 32
_NW = 32
_BPW = _BATCH // _NW
_GROUPS = _BPW // 16


def _body(samples_hbm, user_emb, game_emb, user_bias, game_bias, out_hbm,
          samp_v, idxu_v, idxg_v, idxub_v, idxgb_v,
          urows_v, grows_v, ubrows_v, gbrows_v, out_v, sem):
    wid = lax.axis_index("s") * 2 + lax.axis_index("c")
    base = wid * _BPW

    pltpu.sync_copy(samples_hbm.at[pl.ds(base, _BPW)], samp_v)

    def deint(i, _):
        ids = lax.iota(jnp.int32, 16) + i * 16
        zero = jnp.zeros((16,), jnp.int32)
        one = jnp.ones((16,), jnp.int32)
        u = plsc.load_gather(samp_v, [ids, zero])
        g = plsc.load_gather(samp_v, [ids, one])
        idxu_v[pl.ds(i * 16, 16)] = u
        idxg_v[pl.ds(i * 16, 16)] = g
        idxub_v[pl.ds(i * 16, 16)] = u >> 4
        idxgb_v[pl.ds(i * 16, 16)] = g >> 4
        return 0
    lax.fori_loop(0, _GROUPS, deint, 0)

    c1 = pltpu.async_copy(user_emb.at[idxu_v], urows_v, sem)
    c2 = pltpu.async_copy(game_emb.at[idxg_v], grows_v, sem)
    c3 = pltpu.async_copy(user_bias.at[idxub_v], ubrows_v, sem)
    c4 = pltpu.async_copy(game_bias.at[idxgb_v], gbrows_v, sem)
    c1.wait()
    c2.wait()
    c3.wait()
    c4.wait()

    def dot_group(i, _):
        ids = lax.iota(jnp.int32, 16) + i * 16
        lane_u = idxu_v[pl.ds(i * 16, 16)] & 15
        lane_g = idxg_v[pl.ds(i * 16, 16)] & 15
        acc = plsc.load_gather(ubrows_v, [ids, lane_u]) + \
            plsc.load_gather(gbrows_v, [ids, lane_g])
        lanes = lax.iota(jnp.int32, 16)
        for t in range(_DIM):
            dd = (lanes + t) & (_DIM - 1)
            acc = acc + (plsc.load_gather(urows_v, [ids, dd]) *
                         plsc.load_gather(grows_v, [ids, dd]))
        out_v[pl.ds(i * 16, 16)] = acc
        return 0
    lax.fori_loop(0, _GROUPS, dot_group, 0)

    pltpu.sync_copy(out_v, out_hbm.at[pl.ds(base, _BPW)])


@jax.jit
def _run(samples, user_emb, game_emb, user_bias16, game_bias16):
    mesh = plsc.VectorSubcoreMesh(core_axis_name="c", subcore_axis_name="s")
    f = functools.partial(
        pl.kernel,
        mesh=mesh,
        out_type=jax.ShapeDtypeStruct((_BATCH,), jnp.float32),
        compiler_params=pltpu.CompilerParams(
            needs_layout_passes=False, use_tc_tiling_on_sc=False),
        scratch_types=[
            pltpu.VMEM((_BPW, 2), jnp.int32),
            pltpu.VMEM((_BPW,), jnp.int32),
            pltpu.VMEM((_BPW,), jnp.int32),
            pltpu.VMEM((_BPW,), jnp.int32),
            pltpu.VMEM((_BPW,), jnp.int32),
            pltpu.VMEM((_BPW, _DIM), jnp.float32),
            pltpu.VMEM((_BPW, _DIM), jnp.float32),
            pltpu.VMEM((_BPW, 16), jnp.float32),
            pltpu.VMEM((_BPW, 16), jnp.float32),
            pltpu.VMEM((_BPW,), jnp.float32),
            pltpu.SemaphoreType.DMA,
        ],
    )(_body)
    return f(samples, user_emb, game_emb, user_bias16, game_bias16)


def kernel(samples, user_emb, game_emb, user_bias, game_bias):
    samples_i32 = samples.astype(jnp.int32)
    ub16 = user_bias.reshape(user_bias.shape[0] // 16, 16)
    gb16 = game_bias.reshape(game_bias.shape[0] // 16, 16)
    return _run(samples_i32, user_emb, game_emb, ub16, gb16)
